# Initial kernel scaffold; baseline (speedup 1.0000x reference)
import jax
import jax.numpy as jnp
from jax.experimental import pallas as pl


def kernel(solvent_x, solvent_edge_index, solvent_batch, solvent_y, solvent_ap, solvent_bp, solvent_topopsa, solvent_inter_hb, solute_x, solute_edge_index, solute_batch, solute_topopsa, solute_inter_hb, T_x, W1, b1, W2, b2, proj_W, proj_b, en1_W, en1_b, en2_W, en2_b, root_W, conv_b, gru_Wi, gru_bi, gru_Wh, gru_bh, mlp1_W, mlp1_b, mlp2_W, mlp2_b, mlp3_W, mlp3_b):
    raise NotImplementedError("write your pallas kernel here")


# speedup vs baseline: 10.1044x; 10.1044x over previous
import functools

import jax
import jax.numpy as jnp
from jax import lax
from jax.experimental import pallas as pl
from jax.experimental.pallas import tpu as pltpu
from jax.experimental.pallas import tpu_sc as plsc

F32 = jnp.float32
I32 = jnp.int32

N = 10000
D = 128
B = 512
E = 320000
NT = 16
NC = 2
RPT = 640
ACC = NT * RPT
CH = 128
NCH = 160
GC = 16
NG = NCH // GC
EPAD = NT * NCH * CH
BCH = 5
SEGP = 528
RT = ACC // 128

_mesh = plsc.VectorSubcoreMesh(core_axis_name="c", subcore_axis_name="s")


@functools.partial(
    pl.kernel,
    mesh=_mesh,
    out_type=[jax.ShapeDtypeStruct((NC * ACC,), F32),
              jax.ShapeDtypeStruct((NC * SEGP,), F32)],
    scratch_types=[
        pltpu.VMEM((NCH, CH), I32),
        pltpu.VMEM((BCH, CH), I32),
        pltpu.VMEM((CH,), F32),
        pltpu.VMEM((RPT,), F32),
        pltpu.VMEM_SHARED((ACC,), F32),
        pltpu.VMEM_SHARED((SEGP,), F32),
    ],
)
def _deg_counts(dst_hbm, bat_hbm, deg_hbm, cnt_hbm,
                dst_v, bat_v, ones_v, zbuf, deg_sh, cnt_sh):
    c = lax.axis_index("c")
    s = lax.axis_index("s")
    pltpu.sync_copy(dst_hbm.at[c, s], dst_v)
    pltpu.sync_copy(bat_hbm.at[c, s], bat_v)
    for i in range(CH // 16):
        ones_v[pl.ds(i * 16, 16)] = jnp.full((16,), 1.0, F32)
    for i in range(RPT // 16):
        zbuf[pl.ds(i * 16, 16)] = jnp.zeros((16,), F32)
    pltpu.sync_copy(zbuf, deg_sh.at[pl.ds(s * RPT, RPT)])

    @pl.when(s == 0)
    def _():
        pltpu.sync_copy(zbuf.at[pl.ds(0, SEGP)], cnt_sh)

    plsc.subcore_barrier()

    def ebody(j, carry):
        pltpu.sync_copy(ones_v, deg_sh.at[dst_v.at[j]], add=True)
        return carry

    lax.fori_loop(0, NCH, ebody, 0)

    def bbody(j, carry):
        pltpu.sync_copy(ones_v, cnt_sh.at[bat_v.at[j]], add=True)
        return carry

    lax.fori_loop(0, BCH, bbody, 0)

    plsc.subcore_barrier()
    pltpu.sync_copy(deg_sh.at[pl.ds(s * RPT, RPT)], zbuf)
    pltpu.sync_copy(zbuf, deg_hbm.at[pl.ds(c * ACC + s * RPT, RPT)])

    @pl.when(s == 0)
    def _():
        pltpu.sync_copy(cnt_sh, zbuf.at[pl.ds(0, SEGP)])
        pltpu.sync_copy(zbuf.at[pl.ds(0, SEGP)], cnt_hbm.at[pl.ds(c * SEGP, SEGP)])


@functools.partial(
    pl.kernel,
    mesh=_mesh,
    out_type=jax.ShapeDtypeStruct((NC, ACC, D), F32),
    scratch_types=[
        pltpu.VMEM((GC, CH), I32),
        pltpu.VMEM((GC, CH), I32),
        pltpu.VMEM((CH, D), F32),
        pltpu.VMEM((CH, D), F32),
        pltpu.VMEM_SHARED((ACC, D), F32),
        pltpu.SemaphoreType.DMA,
        pltpu.SemaphoreType.DMA,
    ],
)
def _edge_scatter(y_hbm, src_hbm, dst_hbm, out_hbm,
                  src_v, dst_v, buf0, buf1, acc_sh, sem0, sem1):
    c = lax.axis_index("c")
    s = lax.axis_index("s")

    def zrow(j, carry):
        for i in range(D // 16):
            buf0[j, pl.ds(i * 16, 16)] = jnp.zeros((16,), F32)
        return carry

    lax.fori_loop(0, CH, zrow, 0)
    for t in range(RPT // CH):
        pltpu.sync_copy(buf0, acc_sh.at[pl.ds(s * RPT + t * CH, CH)])
    plsc.subcore_barrier()

    def group(g, carry):
        pltpu.sync_copy(src_hbm.at[c, s, pl.ds(g * GC, GC)], src_v)
        pltpu.sync_copy(dst_hbm.at[c, s, pl.ds(g * GC, GC)], dst_v)
        pltpu.async_copy(y_hbm.at[src_v.at[0]], buf0, sem0)

        def pair(jj, carry2):
            j = jj * 2
            pltpu.async_copy(y_hbm.at[src_v.at[j + 1]], buf1, sem1)
            pltpu.make_async_copy(y_hbm.at[src_v.at[j]], buf0, sem0).wait()
            pltpu.sync_copy(buf0, acc_sh.at[dst_v.at[j]], add=True)

            @pl.when(jj < GC // 2 - 1)
            def _():
                pltpu.async_copy(y_hbm.at[src_v.at[j + 2]], buf0, sem0)

            pltpu.make_async_copy(y_hbm.at[src_v.at[j + 1]], buf1, sem1).wait()
            pltpu.sync_copy(buf1, acc_sh.at[dst_v.at[j + 1]], add=True)
            return carry2

        lax.fori_loop(0, GC // 2, pair, 0)
        return carry

    lax.fori_loop(0, NG, group, 0)

    plsc.subcore_barrier()
    for t in range(RPT // CH):
        pltpu.sync_copy(acc_sh.at[pl.ds(s * RPT + t * CH, CH)], buf0)
        pltpu.sync_copy(buf0, out_hbm.at[c, pl.ds(s * RPT + t * CH, CH)])


def _y_body(x_ref, w_ref, dinv_ref, y_ref):
    y_ref[...] = (dinv_ref[0] *
                  jnp.dot(x_ref[0], w_ref[...], preferred_element_type=F32))[None]


_y_kernel = pl.pallas_call(
    _y_body,
    grid=(NC, RT),
    in_specs=[pl.BlockSpec((1, 128, D), lambda g, i: (g, i, 0)),
              pl.BlockSpec((D, D), lambda g, i: (0, 0)),
              pl.BlockSpec((1, 128, 1), lambda g, i: (g, i, 0))],
    out_specs=pl.BlockSpec((1, 128, D), lambda g, i: (g, i, 0)),
    out_shape=jax.ShapeDtypeStruct((NC, ACC, D), F32),
)


def _layer2_body(acc_ref, y_ref, dinv_ref, b_ref, w_ref, y2_ref):
    x1 = jnp.maximum(dinv_ref[0] * (acc_ref[0] + y_ref[0]) + b_ref[...], 0.0)
    y2_ref[...] = (dinv_ref[0] *
                   jnp.dot(x1, w_ref[...], preferred_element_type=F32))[None]


_layer2 = pl.pallas_call(
    _layer2_body,
    grid=(NC, RT),
    in_specs=[pl.BlockSpec((1, 128, D), lambda g, i: (g, i, 0)),
              pl.BlockSpec((1, 128, D), lambda g, i: (g, i, 0)),
              pl.BlockSpec((1, 128, 1), lambda g, i: (g, i, 0)),
              pl.BlockSpec((1, D), lambda g, i: (0, 0)),
              pl.BlockSpec((D, D), lambda g, i: (0, 0))],
    out_specs=pl.BlockSpec((1, 128, D), lambda g, i: (g, i, 0)),
    out_shape=jax.ShapeDtypeStruct((NC, ACC, D), F32),
)


def _segsum_body(acc_ref, y_ref, dinv_ref, b_ref, bat_ref, out_ref):
    i = pl.program_id(1)
    x2 = jnp.maximum(dinv_ref[0] * (acc_ref[0] + y_ref[0]) + b_ref[...], 0.0)
    bt = bat_ref[0, 0, :]
    seg = lax.broadcasted_iota(I32, (B, 128), 0)
    oh = (seg == bt[None, :]).astype(F32)
    contrib = jnp.dot(oh, x2, preferred_element_type=F32)

    @pl.when(i == 0)
    def _():
        out_ref[...] = contrib[None]

    @pl.when(i != 0)
    def _():
        out_ref[...] += contrib[None]


_segsum = pl.pallas_call(
    _segsum_body,
    grid=(NC, RT),
    in_specs=[pl.BlockSpec((1, 128, D), lambda g, i: (g, i, 0)),
              pl.BlockSpec((1, 128, D), lambda g, i: (g, i, 0)),
              pl.BlockSpec((1, 128, 1), lambda g, i: (g, i, 0)),
              pl.BlockSpec((1, D), lambda g, i: (0, 0)),
              pl.BlockSpec((1, 1, 128), lambda g, i: (g * RT + i, 0, 0))],
    out_specs=pl.BlockSpec((1, B, D), lambda g, i: (g, 0, 0)),
    out_shape=jax.ShapeDtypeStruct((NC, B, D), F32),
)


def _final_body(xgsum_ref, cinv_ref, ap_ref, bp_ref, tp1_ref, tp2_ref,
                hbv_ref, hbu_ref, tx_ref, wpx_ref, wpu_ref, pb_ref,
                e1w_ref, e1b_ref, w3_ref, bm_ref, rw_ref, cb_ref,
                wi_ref, bi_ref, wh_ref, bh_ref,
                m1x_ref, m1t_ref, m1b_ref, m2w_ref, m2b_ref, m3w_ref, m3b_ref,
                out_ref):
    xg1 = xgsum_ref[0] * cinv_ref[0]
    xg2 = xgsum_ref[1] * cinv_ref[1]
    u1 = (ap_ref[...] * wpu_ref[0:1, :] + bp_ref[...] * wpu_ref[1:2, :]
          + tp1_ref[...] * wpu_ref[2:3, :])
    u2 = (ap_ref[...] * wpu_ref[0:1, :] + bp_ref[...] * wpu_ref[1:2, :]
          + tp2_ref[...] * wpu_ref[2:3, :])
    h0t = jnp.maximum(
        jnp.dot(xg1, wpx_ref[...], preferred_element_type=F32) + u1 + pb_ref[...], 0.0)
    h0b = jnp.maximum(
        jnp.dot(xg2, wpx_ref[...], preferred_element_type=F32) + u2 + pb_ref[...], 0.0)
    ht = jnp.maximum(hbv_ref[...] * e1w_ref[...] + e1b_ref[...], 0.0)
    hb = jnp.maximum(hbu_ref[...] * e1w_ref[...] + e1b_ref[...], 0.0)
    rt = jnp.zeros((B, D), F32)
    rb = jnp.zeros((B, D), F32)
    s2 = jnp.zeros((B, D), F32)
    for k in range(32):
        wk = w3_ref[k]
        gt = jnp.dot(h0t, wk, preferred_element_type=F32)
        gb = jnp.dot(h0b, wk, preferred_element_type=F32)
        rt = rt + ht[:, k:k + 1] * gt
        rb = rb + hb[:, k:k + 1] * gb
        s2 = s2 + ht[:, k:k + 1] * gb
    qs = jnp.dot(h0t + h0b, bm_ref[...], preferred_element_type=F32)
    agg_t = rt + s2 + qs
    agg_b = rt + rb + qs

    def gru(h0, agg):
        m = jnp.maximum(
            jnp.dot(h0, rw_ref[...], preferred_element_type=F32) + agg + cb_ref[...], 0.0)
        gi = lax.dot_general(m, wi_ref[...], (((1,), (1,)), ((), ())),
                             preferred_element_type=F32) + bi_ref[...]
        gh = lax.dot_general(h0, wh_ref[...], (((1,), (1,)), ((), ())),
                             preferred_element_type=F32) + bh_ref[...]
        r = jax.nn.sigmoid(gi[:, :D] + gh[:, :D])
        z = jax.nn.sigmoid(gi[:, D:2 * D] + gh[:, D:2 * D])
        n = jnp.tanh(gi[:, 2 * D:] + r * gh[:, 2 * D:])
        return (1.0 - z) * n + z * h0

    o_t = gru(h0t, agg_t)
    o_b = gru(h0b, agg_b)
    xcat = jnp.concatenate([o_t, o_b], axis=1)
    t_norm = (tx_ref[...] + 273.15 - (-60.0 + 273.15)) / ((289.3 + 273.15) - (-60.0 + 273.15))
    o1 = jnp.maximum(
        jnp.dot(xcat, m1x_ref[...], preferred_element_type=F32)
        + t_norm * m1t_ref[...] + m1b_ref[...], 0.0)
    o2 = jnp.maximum(
        jnp.dot(o1, m2w_ref[...], preferred_element_type=F32) + m2b_ref[...], 0.0)
    out_ref[...] = jnp.dot(o2, m3w_ref[...], preferred_element_type=F32) + m3b_ref[...]


_final = pl.pallas_call(
    _final_body,
    out_shape=jax.ShapeDtypeStruct((B, 1), F32),
)


def kernel(solvent_x, solvent_edge_index, solvent_batch, solvent_y, solvent_ap,
           solvent_bp, solvent_topopsa, solvent_inter_hb, solute_x,
           solute_edge_index, solute_batch, solute_topopsa, solute_inter_hb,
           T_x, W1, b1, W2, b2, proj_W, proj_b, en1_W, en1_b, en2_W, en2_b,
           root_W, conv_b, gru_Wi, gru_bi, gru_Wh, gru_bh, mlp1_W, mlp1_b,
           mlp2_W, mlp2_b, mlp3_W, mlp3_b):
    npad_e = EPAD - E

    def prep(ei, g):
        src = jnp.concatenate([ei[0] + g * ACC, jnp.full((npad_e,), g * ACC, I32)])
        dst = jnp.concatenate([ei[1], jnp.full((npad_e,), N, I32)])
        return src.reshape(NT, NCH, CH), dst.reshape(NT, NCH, CH)

    s0, d0 = prep(solvent_edge_index, 0)
    s1, d1 = prep(solute_edge_index, 1)
    src_i = jnp.stack([s0, s1])
    dst_i = jnp.stack([d0, d1])

    def prepb(bat):
        return jnp.concatenate(
            [bat, jnp.full((NT * BCH * CH - N,), B, I32)]).reshape(NT, BCH, CH)

    bat_i = jnp.stack([prepb(solvent_batch), prepb(solute_batch)])

    deg, cnt = _deg_counts(dst_i, bat_i)
    cnt = cnt.reshape(NC, SEGP)
    dinv = lax.rsqrt(deg + 1.0).reshape(NC, ACC, 1)
    x_all = jnp.stack([jnp.pad(solvent_x, ((0, ACC - N), (0, 0))),
                       jnp.pad(solute_x, ((0, ACC - N), (0, 0)))])
    y1 = _y_kernel(x_all, W1, dinv)
    acc1 = _edge_scatter(y1.reshape(NC * ACC, D), src_i, dst_i)
    y2 = _layer2(acc1, y1, dinv, b1.reshape(1, D), W2)
    acc2 = _edge_scatter(y2.reshape(NC * ACC, D), src_i, dst_i)

    def prepbr(bat):
        return jnp.concatenate([bat, jnp.full((ACC - N,), B, I32)])

    bat_r = jnp.stack([prepbr(solvent_batch),
                       prepbr(solute_batch)]).reshape(NC * RT, 1, CH)
    xgsum = _segsum(acc2, y2, dinv, b2.reshape(1, D), bat_r)

    cinv = (1.0 / jnp.maximum(cnt[:, :B], 1.0)).reshape(NC, B, 1)
    out = _final(
        xgsum, cinv,
        solvent_ap.reshape(B, 1), solvent_bp.reshape(B, 1),
        solvent_topopsa.reshape(B, 1), solute_topopsa.reshape(B, 1),
        solvent_inter_hb.reshape(B, 1), solute_inter_hb.reshape(B, 1),
        T_x.reshape(B, 1),
        proj_W[:D], jnp.pad(proj_W[D:], ((0, 5), (0, 0))), proj_b.reshape(1, D),
        en1_W, en1_b.reshape(1, 32), en2_W.reshape(32, D, D),
        en2_b.reshape(D, D), root_W, conv_b.reshape(1, D),
        gru_Wi, gru_bi.reshape(1, 3 * D), gru_Wh, gru_bh.reshape(1, 3 * D),
        mlp1_W[:2 * D], mlp1_W[2 * D:], mlp1_b.reshape(1, 2 * D),
        mlp2_W, mlp2_b.reshape(1, D), mlp3_W, mlp3_b.reshape(1, 1))
    return out
